# Initial kernel scaffold; baseline (speedup 1.0000x reference)
import jax
import jax.numpy as jnp
from jax.experimental import pallas as pl


def kernel(x, edge_index, edge_attr, batch, W_in, b_in, We, be, W1, b1, g1, bt1, W2, b2, g2, bt2, Wc1, bc1, gc, btc, Wc2, bc2):
    raise NotImplementedError("write your pallas kernel here")


# speedup vs baseline: 2.0165x; 2.0165x over previous
import functools

import numpy as np

import jax
import jax.numpy as jnp
from jax import lax
from jax.experimental import pallas as pl
from jax.experimental.pallas import tpu as pltpu
from jax.experimental.pallas import tpu_sc as plsc

N = 10000
E = 320000
D_IN = 128
H = 256
HH = 128
ED = 16
L = 3
G = 64
C = 10

NSUB = 16
EPS_ = E // NSUB
CHUNK = 128
NFULL = EPS_ // CHUNK
TAIL = EPS_ - NFULL * CHUNK
ROWS_A = 624
ROWS_B = N - 15 * ROWS_A

_SQC = float(np.sqrt(np.float64(np.float32(1.0 + 1e-5))))


def _h0_body(x_ref, w_ref, b_ref, oa_ref, ob_ref):
    y = jnp.dot(x_ref[...], w_ref[...], preferred_element_type=jnp.float32)
    y = y + b_ref[...]
    oa_ref[...] = y[:, :HH]
    ob_ref[...] = y[:, HH:]


def _h0(x, w, b):
    bs = 1000
    return pl.pallas_call(
        _h0_body,
        grid=(N // bs,),
        in_specs=[
            pl.BlockSpec((bs, D_IN), lambda i: (i, 0)),
            pl.BlockSpec((D_IN, H), lambda i: (0, 0)),
            pl.BlockSpec((1, H), lambda i: (0, 0)),
        ],
        out_specs=[
            pl.BlockSpec((bs, HH), lambda i: (i, 0)),
            pl.BlockSpec((bs, HH), lambda i: (i, 0)),
        ],
        out_shape=[
            jax.ShapeDtypeStruct((N, HH), jnp.float32),
            jax.ShapeDtypeStruct((N, HH), jnp.float32),
        ],
    )(x, w, b)


def _ea_body(ea_ref, we_ref, be_ref, oa_ref, ob_ref):
    m = jnp.dot(ea_ref[...], we_ref[0], preferred_element_type=jnp.float32)
    m = m + be_ref[0]
    oa_ref[0] = m[:, :HH]
    ob_ref[0] = m[:, HH:]


def _ea_all(edge_attr, We, be):
    bs = 4000
    return pl.pallas_call(
        _ea_body,
        grid=(L, E // bs),
        in_specs=[
            pl.BlockSpec((bs, ED), lambda l, e: (e, 0)),
            pl.BlockSpec((1, ED, H), lambda l, e: (l, 0, 0)),
            pl.BlockSpec((1, 1, H), lambda l, e: (l, 0, 0)),
        ],
        out_specs=[
            pl.BlockSpec((1, bs, HH), lambda l, e: (l, e, 0)),
            pl.BlockSpec((1, bs, HH), lambda l, e: (l, e, 0)),
        ],
        out_shape=[
            jax.ShapeDtypeStruct((L, E, HH), jnp.float32),
            jax.ShapeDtypeStruct((L, E, HH), jnp.float32),
        ],
    )(edge_attr, We, be.reshape(L, 1, H))


def _sc_body(ha_ref, hb_ref, eaa_ref, eab_ref, src_ref, dst_ref,
             oa_ref, ob_ref,
             aggr_sp, src_v, dst_v, hv, eav, src_t, dst_t, ht, eat, sem):
    cid = lax.axis_index("c")
    sid = lax.axis_index("s")

    zv = jnp.zeros((16,), jnp.float32)

    def _zrow(r, carry):
        for j in range(HH // 16):
            hv[r, pl.ds(j * 16, 16)] = zv
        return carry

    lax.fori_loop(0, CHUNK, _zrow, 0)
    base = pl.multiple_of(sid * ROWS_A, 8)
    for t in range(4):
        pltpu.sync_copy(hv, aggr_sp.at[pl.ds(base + t * 128, 128)])

    @pl.when(sid < NSUB - 1)
    def _():
        pltpu.sync_copy(hv.at[pl.ds(0, ROWS_A - 512)],
                        aggr_sp.at[pl.ds(base + 512, ROWS_A - 512)])

    @pl.when(sid == NSUB - 1)
    def _():
        pltpu.sync_copy(hv.at[pl.ds(0, ROWS_B - 512)],
                        aggr_sp.at[pl.ds(base + 512, ROWS_B - 512)])

    plsc.subcore_barrier()

    ebase = sid * EPS_

    def _edges(idx_v, jdx_v, gat, eab_buf, eb, k):
        pltpu.sync_copy(src_ref.at[pl.ds(eb, k)], idx_v)
        pltpu.sync_copy(dst_ref.at[pl.ds(eb, k)], jdx_v)

        @pl.when(cid == 0)
        def _():
            d = pltpu.async_copy(ha_ref.at[idx_v], gat, sem)
            pltpu.sync_copy(eaa_ref.at[pl.ds(eb, k)], eab_buf)
            d.wait()

        @pl.when(cid == 1)
        def _():
            d = pltpu.async_copy(hb_ref.at[idx_v], gat, sem)
            pltpu.sync_copy(eab_ref.at[pl.ds(eb, k)], eab_buf)
            d.wait()

        def _row(r, carry):
            for j in range(HH // 16):
                s = pl.ds(j * 16, 16)
                gat[r, s] = jnp.maximum(gat[r, s] + eab_buf[r, s], 0.0)
            return carry

        lax.fori_loop(0, k, _row, 0)
        pltpu.sync_copy(gat, aggr_sp.at[jdx_v], add=True)

    def _chunk(i, carry):
        eb = pl.multiple_of(ebase + i * CHUNK, 8)
        _edges(src_v, dst_v, hv, eav, eb, CHUNK)
        return carry

    lax.fori_loop(0, NFULL, _chunk, 0)
    _edges(src_t, dst_t, ht, eat, ebase + NFULL * CHUNK, TAIL)

    plsc.subcore_barrier()

    def _drain(out_ref):
        @pl.when(sid < NSUB - 1)
        def _():
            pltpu.sync_copy(aggr_sp.at[pl.ds(base, ROWS_A)],
                            out_ref.at[pl.ds(base, ROWS_A)])

        @pl.when(sid == NSUB - 1)
        def _():
            pltpu.sync_copy(aggr_sp.at[pl.ds(base, ROWS_B)],
                            out_ref.at[pl.ds(base, ROWS_B)])

    @pl.when(cid == 0)
    def _():
        _drain(oa_ref)

    @pl.when(cid == 1)
    def _():
        _drain(ob_ref)


@functools.cache
def _sc_aggr_fn():
    return pl.kernel(
        _sc_body,
        out_type=(
            jax.ShapeDtypeStruct((N, HH), jnp.float32),
            jax.ShapeDtypeStruct((N, HH), jnp.float32),
        ),
        mesh=plsc.VectorSubcoreMesh(core_axis_name="c", subcore_axis_name="s"),
        scratch_types=[
            pltpu.VMEM_SHARED((N, HH), jnp.float32),
            pltpu.VMEM((CHUNK,), jnp.int32),
            pltpu.VMEM((CHUNK,), jnp.int32),
            pltpu.VMEM((CHUNK, HH), jnp.float32),
            pltpu.VMEM((CHUNK, HH), jnp.float32),
            pltpu.VMEM((TAIL,), jnp.int32),
            pltpu.VMEM((TAIL,), jnp.int32),
            pltpu.VMEM((TAIL, HH), jnp.float32),
            pltpu.VMEM((TAIL, HH), jnp.float32),
            pltpu.SemaphoreType.DMA,
        ],
    )


def _sc_aggr(ha, hb, eaa, eab, src, dst):
    return _sc_aggr_fn()(ha, hb, eaa, eab, src, dst)


def _bn_relu(y, g, bt):
    return jnp.maximum(g * (y / _SQC) + bt, 0.0)


def _mlp_body(ha_ref, hb_ref, aa_ref, ab_ref, w1_ref, b1_ref, g1_ref, t1_ref,
              w2_ref, b2_ref, g2_ref, t2_ref, oa_ref, ob_ref):
    z = jnp.concatenate(
        [ha_ref[...] + aa_ref[...], hb_ref[...] + ab_ref[...]], axis=-1)
    z = jnp.dot(z, w1_ref[...], preferred_element_type=jnp.float32)
    z = _bn_relu(z + b1_ref[...], g1_ref[...], t1_ref[...])
    z = jnp.dot(z, w2_ref[...], preferred_element_type=jnp.float32)
    z = _bn_relu(z + b2_ref[...], g2_ref[...], t2_ref[...])
    oa_ref[...] = z[:, :HH]
    ob_ref[...] = z[:, HH:]


def _mlp(ha, hb, aa, ab, w1, b1, g1, t1, w2, b2, g2, t2):
    bs = 1000
    vspec = pl.BlockSpec((1, H), lambda i: (0, 0))
    return pl.pallas_call(
        _mlp_body,
        grid=(N // bs,),
        in_specs=[
            pl.BlockSpec((bs, HH), lambda i: (i, 0)),
            pl.BlockSpec((bs, HH), lambda i: (i, 0)),
            pl.BlockSpec((bs, HH), lambda i: (i, 0)),
            pl.BlockSpec((bs, HH), lambda i: (i, 0)),
            pl.BlockSpec((H, H), lambda i: (0, 0)),
            vspec, vspec, vspec,
            pl.BlockSpec((H, H), lambda i: (0, 0)),
            vspec, vspec, vspec,
        ],
        out_specs=[
            pl.BlockSpec((bs, HH), lambda i: (i, 0)),
            pl.BlockSpec((bs, HH), lambda i: (i, 0)),
        ],
        out_shape=[
            jax.ShapeDtypeStruct((N, HH), jnp.float32),
            jax.ShapeDtypeStruct((N, HH), jnp.float32),
        ],
    )(ha, hb, aa, ab, w1, b1, g1, t1, w2, b2, g2, t2)


def _pool_body(h1a, h1b, h2a, h2b, h3a, h3b, br_ref, bc_ref,
               wc1_ref, bc1_ref, gc_ref, tc_ref, wc2_ref, bc2_ref, out_ref,
               sums, cnt, mx):
    i = pl.program_id(0)

    @pl.when(i == 0)
    def _():
        sums[...] = jnp.zeros_like(sums)
        cnt[...] = jnp.zeros_like(cnt)
        mx[...] = jnp.zeros_like(mx)

    jk = jnp.concatenate(
        [h1a[...], h1b[...], h2a[...], h2b[...], h3a[...], h3b[...]], axis=-1)
    brow = br_ref[0]
    bcol = bc_ref[0]
    gids = lax.broadcasted_iota(jnp.int32, (G, brow.shape[1]), 0)
    m = (gids == brow).astype(jnp.float32)
    sums[...] += jnp.dot(m, jk, preferred_element_type=jnp.float32,
                         precision=lax.Precision.HIGHEST)
    cnt[...] += jnp.dot(m, jnp.ones((brow.shape[1], 128), jnp.float32),
                        preferred_element_type=jnp.float32,
                        precision=lax.Precision.HIGHEST)
    for g in range(G):
        mg = jnp.max(jnp.where(bcol == g, jk, 0.0), axis=0, keepdims=True)
        mx[pl.ds(g, 1), :] = jnp.maximum(mx[pl.ds(g, 1), :], mg)

    @pl.when(i == pl.num_programs(0) - 1)
    def _():
        c = jnp.maximum(cnt[:, 0:1], 1.0)
        z = jnp.concatenate([sums[...] / c, mx[...]], axis=-1)
        z = jnp.dot(z, wc1_ref[...], preferred_element_type=jnp.float32)
        z = _bn_relu(z + bc1_ref[...], gc_ref[...], tc_ref[...])
        z = jnp.dot(z, wc2_ref[...], preferred_element_type=jnp.float32)
        out_ref[...] = z + bc2_ref[...]


def _pool_head(hs, batch, wc1, bc1, gcv, tcv, wc2, bc2):
    bs = 1000
    br = batch.reshape(N // bs, 1, bs)
    bc = batch.reshape(N // bs, bs, 1)
    hspecs = [pl.BlockSpec((bs, HH), lambda i: (i, 0)) for _ in range(6)]
    vspec = pl.BlockSpec((1, H), lambda i: (0, 0))
    return pl.pallas_call(
        _pool_body,
        grid=(N // bs,),
        in_specs=hspecs + [
            pl.BlockSpec((1, 1, bs), lambda i: (i, 0, 0)),
            pl.BlockSpec((1, bs, 1), lambda i: (i, 0, 0)),
            pl.BlockSpec((2 * L * H, H), lambda i: (0, 0)),
            vspec, vspec, vspec,
            pl.BlockSpec((H, C), lambda i: (0, 0)),
            pl.BlockSpec((1, C), lambda i: (0, 0)),
        ],
        out_specs=pl.BlockSpec((G, C), lambda i: (0, 0)),
        out_shape=jax.ShapeDtypeStruct((G, C), jnp.float32),
        scratch_shapes=[
            pltpu.VMEM((G, L * H), jnp.float32),
            pltpu.VMEM((G, 128), jnp.float32),
            pltpu.VMEM((G, L * H), jnp.float32),
        ],
    )(*hs, br, bc, wc1, bc1, gcv, tcv, wc2, bc2)


def kernel(x, edge_index, edge_attr, batch, W_in, b_in, We, be, W1, b1, g1,
           bt1, W2, b2, g2, bt2, Wc1, bc1, gc, btc, Wc2, bc2):
    src = edge_index[0]
    dst = edge_index[1]

    ha, hb = _h0(x, W_in, b_in.reshape(1, H))
    eaa, eab = _ea_all(edge_attr, We, be)

    houts = []
    for l in range(L):
        aa, ab = _sc_aggr(ha, hb, eaa[l], eab[l], src, dst)
        ha, hb = _mlp(ha, hb, aa, ab,
                      W1[l], b1[l].reshape(1, H), g1[l].reshape(1, H),
                      bt1[l].reshape(1, H),
                      W2[l], b2[l].reshape(1, H), g2[l].reshape(1, H),
                      bt2[l].reshape(1, H))
        houts += [ha, hb]

    return _pool_head(houts, batch, Wc1, bc1.reshape(1, H),
                      gc.reshape(1, H), btc.reshape(1, H),
                      Wc2, bc2.reshape(1, C))
